# baseline (speedup 1.0000x reference)
import functools
import jax
import jax.numpy as jnp
from jax.experimental import pallas as pl
from jax.experimental.pallas import tpu as pltpu

_BN = 400
_ABN = 400


_WIN = 400


def _deg_cast_body(adj_ref, x_ref, w0_ref, abf_ref, d_ref, c_ref, u1_ref, *,
                   bn):
    i = pl.program_id(0)
    n = adj_ref.shape[1]
    blk = adj_ref[...]
    rowsum = jnp.sum(blk, axis=1, keepdims=True)
    start = jnp.minimum((i * bn) // 128 * 128, n - _WIN)
    start = pl.multiple_of(start, 128)
    win = adj_ref[:, pl.ds(start, _WIN)]
    shift = i * bn - start
    rr = jax.lax.broadcasted_iota(jnp.int32, (bn, _WIN), 0)
    cc = jax.lax.broadcasted_iota(jnp.int32, (bn, _WIN), 1)
    diag = jnp.sum(jnp.where(cc == rr + shift, win, 0.0), axis=1, keepdims=True)
    deg = jnp.maximum(rowsum - diag + 1.0, 1.0)
    d = jax.lax.rsqrt(deg)
    d_ref[...] = d
    c_ref[...] = 1.0 - diag
    abf_ref[...] = blk.astype(jnp.bfloat16)
    z = jnp.dot(x_ref[...], w0_ref[...], preferred_element_type=jnp.float32)
    u1_ref[...] = (d * z).astype(jnp.bfloat16)


def _deg_cast(adj, x, w0):
    n = adj.shape[0]
    f, h_out = w0.shape
    ni = n // _BN
    return pl.pallas_call(
        functools.partial(_deg_cast_body, bn=_BN),
        grid=(ni,),
        in_specs=[
            pl.BlockSpec((_BN, n), lambda i: (i, 0)),
            pl.BlockSpec((_BN, f), lambda i: (i, 0)),
            pl.BlockSpec((f, h_out), lambda i: (0, 0)),
        ],
        out_specs=[
            pl.BlockSpec((_BN, n), lambda i: (i, 0)),
            pl.BlockSpec((_BN, 1), lambda i: (i, 0)),
            pl.BlockSpec((_BN, 1), lambda i: (i, 0)),
            pl.BlockSpec((_BN, h_out), lambda i: (i, 0)),
        ],
        out_shape=[
            jax.ShapeDtypeStruct((n, n), jnp.bfloat16),
            jax.ShapeDtypeStruct((n, 1), jnp.float32),
            jax.ShapeDtypeStruct((n, 1), jnp.float32),
            jax.ShapeDtypeStruct((n, h_out), jnp.bfloat16),
        ],
        compiler_params=pltpu.CompilerParams(
            dimension_semantics=("arbitrary",)
        ),
    )(adj, x, w0)


def _mega_body(abf_ref, u1_ref, d_ref, c_ref, bs_ref, w1_ref, w2_ref,
               y_ref, us_ref, *, bn):
    p = pl.program_id(0)
    i = pl.program_id(1)
    b = bs_ref[0]
    d = d_ref[...]
    cc = c_ref[...]
    a = abf_ref[...]

    def step(u_full, ui):
        acc = jnp.dot(a, u_full, preferred_element_type=jnp.float32)
        return d * (acc + cc * ui.astype(jnp.float32)) + b

    @pl.when(p == 0)
    def _():
        y = step(u1_ref[...], u1_ref[pl.ds(i * bn, bn), :])
        z = jnp.dot(y, w1_ref[...], preferred_element_type=jnp.float32)
        us_ref[0, pl.ds(i * bn, bn), :] = (d * z).astype(jnp.bfloat16)

    @pl.when(p == 1)
    def _():
        y = jnp.tanh(step(us_ref[0], us_ref[0, pl.ds(i * bn, bn), :]))
        z = jnp.dot(y, w2_ref[...], preferred_element_type=jnp.float32)
        us_ref[1, pl.ds(i * bn, bn), :] = (d * z).astype(jnp.bfloat16)

    @pl.when(p == 2)
    def _():
        y_ref[...] = jnp.tanh(step(us_ref[1], us_ref[1, pl.ds(i * bn, bn), :]))


def _mega(abf, u1, d, c, bs, w1, w2):
    n, h_out = abf.shape[0], u1.shape[1]
    ni = n // _ABN
    return pl.pallas_call(
        functools.partial(_mega_body, bn=_ABN),
        grid=(3, ni),
        in_specs=[
            pl.BlockSpec((_ABN, n), lambda p, i: (i, 0)),
            pl.BlockSpec((n, h_out), lambda p, i: (0, 0)),
            pl.BlockSpec((_ABN, 1), lambda p, i: (i, 0)),
            pl.BlockSpec((_ABN, 1), lambda p, i: (i, 0)),
            pl.BlockSpec((1, 1, h_out), lambda p, i: (p, 0, 0)),
            pl.BlockSpec(w1.shape, lambda p, i: (0, 0)),
            pl.BlockSpec(w2.shape, lambda p, i: (0, 0)),
        ],
        out_specs=pl.BlockSpec((_ABN, h_out), lambda p, i: (i, 0)),
        out_shape=jax.ShapeDtypeStruct((n, h_out), jnp.float32),
        scratch_shapes=[pltpu.VMEM((2, n, h_out), jnp.bfloat16)],
        compiler_params=pltpu.CompilerParams(
            dimension_semantics=("arbitrary", "arbitrary")
        ),
    )(abf, u1, d, c, bs, w1, w2)


def kernel(x, adj, W0, b0, W1, b1, W2, b2):
    xb = x[0]
    a = adj[0]
    abf, d, c, u1 = _deg_cast(a, xb, W0)
    bs = jnp.stack([b0, b1, b2]).reshape(3, 1, -1)
    h3 = _mega(abf, u1, d, c, bs, W1, W2)
    return h3[None]


# speedup vs baseline: 1.0393x; 1.0393x over previous
import functools
import jax
import jax.numpy as jnp
from jax.experimental import pallas as pl
from jax.experimental.pallas import tpu as pltpu

_BN = 400
_ABN = 1000


_WIN = 400


def _deg_cast_body(adj_ref, x_ref, w0_ref, abf_ref, d_ref, c_ref, u1_ref, *,
                   bn):
    i = pl.program_id(0)
    n = adj_ref.shape[1]
    blk = adj_ref[...]
    rowsum = jnp.sum(blk, axis=1, keepdims=True)
    start = jnp.minimum((i * bn) // 128 * 128, n - _WIN)
    start = pl.multiple_of(start, 128)
    win = adj_ref[:, pl.ds(start, _WIN)]
    shift = i * bn - start
    rr = jax.lax.broadcasted_iota(jnp.int32, (bn, _WIN), 0)
    cc = jax.lax.broadcasted_iota(jnp.int32, (bn, _WIN), 1)
    diag = jnp.sum(jnp.where(cc == rr + shift, win, 0.0), axis=1, keepdims=True)
    deg = jnp.maximum(rowsum - diag + 1.0, 1.0)
    d = jax.lax.rsqrt(deg)
    d_ref[...] = d
    c_ref[...] = 1.0 - diag
    abf_ref[...] = blk.astype(jnp.bfloat16)
    z = jnp.dot(x_ref[...], w0_ref[...], preferred_element_type=jnp.float32)
    u1_ref[...] = (d * z).astype(jnp.bfloat16)


def _deg_cast(adj, x, w0):
    n = adj.shape[0]
    f, h_out = w0.shape
    ni = n // _BN
    return pl.pallas_call(
        functools.partial(_deg_cast_body, bn=_BN),
        grid=(ni,),
        in_specs=[
            pl.BlockSpec((_BN, n), lambda i: (i, 0)),
            pl.BlockSpec((_BN, f), lambda i: (i, 0)),
            pl.BlockSpec((f, h_out), lambda i: (0, 0)),
        ],
        out_specs=[
            pl.BlockSpec((_BN, n), lambda i: (i, 0)),
            pl.BlockSpec((_BN, 1), lambda i: (i, 0)),
            pl.BlockSpec((_BN, 1), lambda i: (i, 0)),
            pl.BlockSpec((_BN, h_out), lambda i: (i, 0)),
        ],
        out_shape=[
            jax.ShapeDtypeStruct((n, n), jnp.bfloat16),
            jax.ShapeDtypeStruct((n, 1), jnp.float32),
            jax.ShapeDtypeStruct((n, 1), jnp.float32),
            jax.ShapeDtypeStruct((n, h_out), jnp.bfloat16),
        ],
        compiler_params=pltpu.CompilerParams(
            dimension_semantics=("arbitrary",)
        ),
    )(adj, x, w0)


def _agg_body(abf_ref, u_ref, d_ref, c_ref, b_ref, y_ref, *, bn, apply_tanh):
    i = pl.program_id(0)
    acc = jnp.dot(abf_ref[...], u_ref[...], preferred_element_type=jnp.float32)
    ui = u_ref[pl.ds(i * bn, bn), :].astype(jnp.float32)
    r = d_ref[...] * (acc + c_ref[...] * ui) + b_ref[...]
    y_ref[...] = jnp.tanh(r) if apply_tanh else r


def _agg_u_body(abf_ref, u_ref, d_ref, c_ref, b_ref, w_ref, un_ref, *, bn,
                apply_tanh):
    i = pl.program_id(0)
    acc = jnp.dot(abf_ref[...], u_ref[...], preferred_element_type=jnp.float32)
    ui = u_ref[pl.ds(i * bn, bn), :].astype(jnp.float32)
    r = d_ref[...] * (acc + c_ref[...] * ui) + b_ref[...]
    y = jnp.tanh(r) if apply_tanh else r
    z = jnp.dot(y, w_ref[...], preferred_element_type=jnp.float32)
    un_ref[...] = (d_ref[...] * z).astype(jnp.bfloat16)


def _agg_u(abf, u, d, c, b, w_next, apply_tanh):
    n, h_out = abf.shape[0], w_next.shape[1]
    ni = n // _ABN
    return pl.pallas_call(
        functools.partial(_agg_u_body, bn=_ABN, apply_tanh=apply_tanh),
        grid=(ni,),
        in_specs=[
            pl.BlockSpec((_ABN, n), lambda i: (i, 0)),
            pl.BlockSpec((n, u.shape[1]), lambda i: (0, 0)),
            pl.BlockSpec((_ABN, 1), lambda i: (i, 0)),
            pl.BlockSpec((_ABN, 1), lambda i: (i, 0)),
            pl.BlockSpec((1, b.shape[1]), lambda i: (0, 0)),
            pl.BlockSpec(w_next.shape, lambda i: (0, 0)),
        ],
        out_specs=pl.BlockSpec((_ABN, h_out), lambda i: (i, 0)),
        out_shape=jax.ShapeDtypeStruct((n, h_out), jnp.bfloat16),
        compiler_params=pltpu.CompilerParams(
            dimension_semantics=("arbitrary",)
        ),
    )(abf, u, d, c, b, w_next)


def _agg(abf, u, d, c, b, apply_tanh):
    n, h_out = abf.shape[0], u.shape[1]
    ni = n // _ABN
    return pl.pallas_call(
        functools.partial(_agg_body, bn=_ABN, apply_tanh=apply_tanh),
        grid=(ni,),
        in_specs=[
            pl.BlockSpec((_ABN, n), lambda i: (i, 0)),
            pl.BlockSpec((n, h_out), lambda i: (0, 0)),
            pl.BlockSpec((_ABN, 1), lambda i: (i, 0)),
            pl.BlockSpec((_ABN, 1), lambda i: (i, 0)),
            pl.BlockSpec((1, h_out), lambda i: (0, 0)),
        ],
        out_specs=pl.BlockSpec((_ABN, h_out), lambda i: (i, 0)),
        out_shape=jax.ShapeDtypeStruct((n, h_out), jnp.float32),
        compiler_params=pltpu.CompilerParams(
            dimension_semantics=("arbitrary",)
        ),
    )(abf, u, d, c, b)


def kernel(x, adj, W0, b0, W1, b1, W2, b2):
    xb = x[0]
    a = adj[0]
    abf, d, c, u1 = _deg_cast(a, xb, W0)
    u2 = _agg_u(abf, u1, d, c, b0.reshape(1, -1), W1, apply_tanh=False)
    u3 = _agg_u(abf, u2, d, c, b1.reshape(1, -1), W2, apply_tanh=True)
    h3 = _agg(abf, u3, d, c, b2.reshape(1, -1), apply_tanh=True)
    return h3[None]
